# Initial kernel scaffold; baseline (speedup 1.0000x reference)
import jax
import jax.numpy as jnp
from jax.experimental import pallas as pl


def kernel(x, edge_index, batch, W1, b1, W2, b2, Wout, bout):
    raise NotImplementedError("write your pallas kernel here")


# speedup vs baseline: 8.1979x; 8.1979x over previous
import functools

import jax
import jax.numpy as jnp
from jax import lax
from jax.experimental import pallas as pl
from jax.experimental.pallas import tpu as pltpu
from jax.experimental.pallas import tpu_sc as plsc

N = 10000
E = 160000
D = 256
H = 512
G = 64
NC = 2
NS = 16
EB = 128
NB = 79
E_PAD = NS * NB * EB
N_PAD = 10240
RPT = N_PAD // NS
NCH = 4
CW = 128
BR = 1280


def _deg_body(dst_hbm, ones_hbm, out_hbm, dst_v, ones_v, acc_sh):
    c = lax.axis_index("c")
    s = lax.axis_index("s")
    for i in range(EB // 16):
        ones_v[pl.ds(i * 16, 16)] = jnp.ones((16,), jnp.float32)
    off = pl.multiple_of(s * RPT, 128)
    pltpu.sync_copy(ones_hbm.at[pl.ds(off, RPT)], acc_sh.at[pl.ds(off, RPT)])
    pltpu.sync_copy(dst_hbm.at[s], dst_v)
    plsc.subcore_barrier()
    lo = c * 40
    hi = 40 + 39 * c

    def body(j, carry):
        pltpu.sync_copy(ones_v, acc_sh.at[dst_v.at[j]], add=True)
        return carry

    lax.fori_loop(lo, hi, body, 0)
    plsc.subcore_barrier()
    pltpu.sync_copy(acc_sh.at[pl.ds(off, RPT)],
                    out_hbm.at[c, 0, pl.ds(off, RPT)])


def _deg_call(dst_p, ones_init):
    mesh = plsc.VectorSubcoreMesh(core_axis_name="c", subcore_axis_name="s")
    return pl.kernel(
        _deg_body,
        out_type=jax.ShapeDtypeStruct((NC, 1, N_PAD), jnp.float32),
        mesh=mesh,
        scratch_types=[
            pltpu.VMEM((NB, EB), jnp.int32),
            pltpu.VMEM((EB,), jnp.float32),
            pltpu.VMEM_SHARED((N_PAD,), jnp.float32),
        ],
    )(dst_p, ones_init)


def _scatter_body(u0, u1, u2, u3, src_hbm, dst_hbm, o0, o1, o2, o3,
                  src_v, dst_v, rows_v, acc_sh, sem):
    c = lax.axis_index("c")
    s = lax.axis_index("s")
    off = pl.multiple_of(lax.axis_index("s") * RPT, 128)
    pltpu.sync_copy(src_hbm.at[s], src_v)
    pltpu.sync_copy(dst_hbm.at[s], dst_v)
    us = (u0, u1, u2, u3)
    os_ = (o0, o1, o2, o3)
    for cv in range(NC):
        @pl.when(c == cv)
        def _():
            for ci in range(NCH // NC):
                ch = (NCH // NC) * cv + ci
                u_ref = us[ch]
                o_ref = os_[ch]
                pltpu.sync_copy(u_ref.at[pl.ds(off, RPT)],
                                acc_sh.at[pl.ds(off, RPT)])
                plsc.subcore_barrier()

                def body(j, carry):
                    pltpu.async_copy(u_ref.at[src_v.at[j]], rows_v, sem).wait()
                    pltpu.sync_copy(rows_v, acc_sh.at[dst_v.at[j]], add=True)
                    return carry

                lax.fori_loop(0, NB, body, 0)
                plsc.subcore_barrier()
                pltpu.sync_copy(acc_sh.at[pl.ds(off, RPT)],
                                o_ref.at[pl.ds(off, RPT)])
                plsc.subcore_barrier()


def _sc_scatter(u_chunks, src_p, dst_p):
    mesh = plsc.VectorSubcoreMesh(core_axis_name="c", subcore_axis_name="s")
    return pl.kernel(
        _scatter_body,
        out_type=[jax.ShapeDtypeStruct((N_PAD, CW), jnp.float32)] * NCH,
        mesh=mesh,
        scratch_types=[
            pltpu.VMEM((NB, EB), jnp.int32),
            pltpu.VMEM((NB, EB), jnp.int32),
            pltpu.VMEM((EB, CW), jnp.float32),
            pltpu.VMEM_SHARED((N_PAD, CW), jnp.float32),
            pltpu.SemaphoreType.DMA,
        ],
    )(*u_chunks, src_p, dst_p)


def _tc1_body(x_ref, degp_ref, w_ref, u0, u1, u2, u3, dinv_ref):
    deg = degp_ref[0] + degp_ref[1] - 1.0
    dinv = lax.rsqrt(deg)
    dinv_ref[...] = dinv
    h = jnp.dot(x_ref[...], w_ref[...], preferred_element_type=jnp.float32)
    u = h * dinv
    for ci, r in enumerate((u0, u1, u2, u3)):
        r[...] = u[:, ci * CW:(ci + 1) * CW]


def _tc1(xp, degp3, W1):
    nb = N_PAD // BR
    return pl.pallas_call(
        _tc1_body,
        grid=(nb,),
        in_specs=[
            pl.BlockSpec((BR, D), lambda i: (i, 0)),
            pl.BlockSpec((NC, BR, 1), lambda i: (0, i, 0)),
            pl.BlockSpec((D, H), lambda i: (0, 0)),
        ],
        out_specs=[pl.BlockSpec((BR, CW), lambda i: (i, 0))] * NCH
        + [pl.BlockSpec((BR, 1), lambda i: (i, 0))],
        out_shape=[jax.ShapeDtypeStruct((N_PAD, CW), jnp.float32)] * NCH
        + [jax.ShapeDtypeStruct((N_PAD, 1), jnp.float32)],
    )(xp, degp3, W1)


def _tc2_body(a0, a1, a2, a3, dinv_ref, b_ref, w_ref, u0, u1, u2, u3):
    a = jnp.concatenate([r[...] for r in (a0, a1, a2, a3)], axis=1)
    z = jnp.maximum(a * dinv_ref[...] + b_ref[...], 0.0)
    u = jnp.dot(z, w_ref[...], preferred_element_type=jnp.float32) * dinv_ref[...]
    for ci, r in enumerate((u0, u1, u2, u3)):
        r[...] = u[:, ci * CW:(ci + 1) * CW]


def _tc2(a_chunks, dinv, b1r, W2):
    nb = N_PAD // BR
    return pl.pallas_call(
        _tc2_body,
        grid=(nb,),
        in_specs=[pl.BlockSpec((BR, CW), lambda i: (i, 0))] * NCH
        + [
            pl.BlockSpec((BR, 1), lambda i: (i, 0)),
            pl.BlockSpec((1, H), lambda i: (0, 0)),
            pl.BlockSpec((H, H), lambda i: (0, 0)),
        ],
        out_specs=[pl.BlockSpec((BR, CW), lambda i: (i, 0))] * NCH,
        out_shape=[jax.ShapeDtypeStruct((N_PAD, CW), jnp.float32)] * NCH,
    )(*a_chunks, dinv, b1r, W2)


def _tc3_body(a0, a1, a2, a3, dinv_ref, b_ref, batch_ref, wout_ref, bout_ref,
              out_ref):
    i = pl.program_id(0)
    a = jnp.concatenate([r[...] for r in (a0, a1, a2, a3)], axis=1)
    z = jnp.maximum(a * dinv_ref[...] + b_ref[...], 0.0)
    y = jnp.dot(z, wout_ref[...], preferred_element_type=jnp.float32)
    gids = lax.broadcasted_iota(jnp.int32, (BR, G), 1)
    oh = (batch_ref[...] == gids).astype(jnp.float32)
    contrib = jnp.sum(oh * y, axis=0).reshape(G, 1)

    @pl.when(i == 0)
    def _():
        out_ref[...] = contrib + bout_ref[...]

    @pl.when(i > 0)
    def _():
        out_ref[...] += contrib


def _tc3(a_chunks, dinv, b2r, batch_p, Wout, boutr):
    nb = N_PAD // BR
    return pl.pallas_call(
        _tc3_body,
        grid=(nb,),
        in_specs=[pl.BlockSpec((BR, CW), lambda i: (i, 0))] * NCH
        + [
            pl.BlockSpec((BR, 1), lambda i: (i, 0)),
            pl.BlockSpec((1, H), lambda i: (0, 0)),
            pl.BlockSpec((BR, 1), lambda i: (i, 0)),
            pl.BlockSpec((H, 1), lambda i: (0, 0)),
            pl.BlockSpec((1, 1), lambda i: (0, 0)),
        ],
        out_specs=pl.BlockSpec((G, 1), lambda i: (0, 0)),
        out_shape=jax.ShapeDtypeStruct((G, 1), jnp.float32),
    )(*a_chunks, dinv, b2r, batch_p, Wout, boutr)


def kernel(x, edge_index, batch, W1, b1, W2, b2, Wout, bout):
    src = edge_index[0].astype(jnp.int32)
    dst = edge_index[1].astype(jnp.int32)
    pad_e = E_PAD - E
    src_p = jnp.concatenate([src, jnp.zeros((pad_e,), jnp.int32)]).reshape(NS, NB, EB)
    dst_p = jnp.concatenate([dst, jnp.full((pad_e,), N, jnp.int32)]).reshape(NS, NB, EB)
    ones_init = jnp.ones((N_PAD,), jnp.float32)
    xp = jnp.pad(x, ((0, N_PAD - N), (0, 0)))
    batch_p = jnp.concatenate(
        [batch.astype(jnp.int32), jnp.full((N_PAD - N,), G, jnp.int32)]
    ).reshape(N_PAD, 1)

    degp = _deg_call(dst_p, ones_init)
    degp3 = degp.reshape(NC, N_PAD, 1)

    *u1c, dinv = _tc1(xp, degp3, W1)
    a1c = _sc_scatter(u1c, src_p, dst_p)
    u2c = _tc2(a1c, dinv, b1.reshape(1, H), W2)
    a2c = _sc_scatter(u2c, src_p, dst_p)
    out = _tc3(a2c, dinv, b2.reshape(1, H), batch_p, Wout, bout.reshape(1, 1))
    return out
